# Initial kernel scaffold; baseline (speedup 1.0000x reference)
import jax
import jax.numpy as jnp
from jax.experimental import pallas as pl


def kernel(feat, W_ih, W_hh, b_ih, b_hh, segment_ids):
    raise NotImplementedError("write your pallas kernel here")


# speedup vs baseline: 6.7564x; 6.7564x over previous
import jax
import jax.numpy as jnp
from jax.experimental import pallas as pl

N = 100000
D = 128
B = 256
N_ITERS = 3
T = 1000
NT = N // T

_NEG_INF = float("-inf")


def _set2set_body(feat_ref, start_ref, end_ref, w_ih_ref, w_hh_ref, bias_ref,
                  out_ref):
    f32 = jnp.float32

    def lstm_step(q_star, h, c):
        gates = (
            jax.lax.dot_general(q_star, w_ih_ref[...],
                                (((1,), (1,)), ((), ())),
                                precision=jax.lax.Precision.HIGHEST,
                                preferred_element_type=f32)
            + jax.lax.dot_general(h, w_hh_ref[...],
                                  (((1,), (1,)), ((), ())),
                                  precision=jax.lax.Precision.HIGHEST,
                                  preferred_element_type=f32)
            + bias_ref[...]
        )
        i_ = jax.nn.sigmoid(gates[:, 0 * D:1 * D])
        f_ = jax.nn.sigmoid(gates[:, 1 * D:2 * D])
        g_ = jnp.tanh(gates[:, 2 * D:3 * D])
        o_ = jax.nn.sigmoid(gates[:, 3 * D:4 * D])
        c_new = f_ * c + i_ * g_
        h_new = o_ * jnp.tanh(c_new)
        return h_new, c_new

    def tile_e(t, q_hi, q_lo):
        rows = pl.ds(t * T, T)
        f_tile = feat_ref[rows, :]
        gidx = (t * T
                + jax.lax.broadcasted_iota(jnp.int32, (T, B), 0))
        onehot = ((gidx >= start_ref[...])
                  & (gidx < end_ref[...]))
        oh_bf = onehot.astype(jnp.bfloat16)
        qg = (
            jax.lax.dot_general(oh_bf, q_hi, (((1,), (0,)), ((), ())),
                                preferred_element_type=f32)
            + jax.lax.dot_general(oh_bf, q_lo, (((1,), (0,)), ((), ())),
                                  preferred_element_type=f32)
        )
        e = jnp.sum(f_tile * qg, axis=1, keepdims=True)
        return e, onehot, f_tile

    h = jnp.zeros((B, D), f32)
    c = jnp.zeros((B, D), f32)
    q_star = jnp.zeros((B, 2 * D), f32)
    ones_col = jnp.ones((T, 1), jnp.bfloat16)
    zero_col = jnp.zeros((T, 1), jnp.bfloat16)

    for _ in range(N_ITERS):
        h, c = lstm_step(q_star, h, c)
        q = h
        q_hi = q.astype(jnp.bfloat16)
        q_lo = (q - q_hi.astype(f32)).astype(jnp.bfloat16)

        def pass1(t, m):
            e, onehot, _ = tile_e(t, q_hi, q_lo)
            tile_m = jnp.max(jnp.where(onehot, e, _NEG_INF), axis=0,
                             keepdims=True)
            return jnp.maximum(m, tile_m)

        m = jax.lax.fori_loop(0, NT, pass1,
                              jnp.full((1, B), _NEG_INF, f32))

        def pass2(t, s_aug):
            e, onehot, f_tile = tile_e(t, q_hi, q_lo)
            m_seg = jnp.sum(jnp.where(onehot, m, 0.0), axis=1,
                            keepdims=True)
            w = jnp.exp(e - m_seg)
            wm = jnp.where(onehot, w, 0.0)
            wm_hi = wm.astype(jnp.bfloat16)
            wm_lo = (wm - wm_hi.astype(f32)).astype(jnp.bfloat16)
            f_hi = jnp.concatenate(
                [f_tile.astype(jnp.bfloat16), ones_col], axis=1)
            f_lo = jnp.concatenate(
                [(f_tile - f_hi[:, :D].astype(f32)).astype(jnp.bfloat16),
                 zero_col], axis=1)
            dims = (((0,), (0,)), ((), ()))
            return (s_aug
                    + jax.lax.dot_general(wm_hi, f_hi, dims,
                                          preferred_element_type=f32)
                    + jax.lax.dot_general(wm_hi, f_lo, dims,
                                          preferred_element_type=f32)
                    + jax.lax.dot_general(wm_lo, f_hi, dims,
                                          preferred_element_type=f32))

        s_aug = jax.lax.fori_loop(0, NT, pass2,
                                  jnp.zeros((B, D + 1), f32))

        s_num = s_aug[:, :D]
        zden = s_aug[:, D:D + 1]
        recip = jnp.where(zden > 0.0, 1.0 / zden, 0.0)
        readout = s_num * recip
        q_star = jnp.concatenate([q, readout], axis=1)

    out_ref[...] = q_star


@jax.jit
def kernel(feat, W_ih, W_hh, b_ih, b_hh, segment_ids):
    seg = segment_ids.astype(jnp.int32)
    offsets = jnp.searchsorted(seg, jnp.arange(B + 1, dtype=jnp.int32),
                               side="left").astype(jnp.int32)
    start = offsets[:B].reshape(1, B)
    end = offsets[1:B + 1].reshape(1, B)
    bias = (b_ih + b_hh).reshape(1, 4 * D)
    return pl.pallas_call(
        _set2set_body,
        out_shape=jax.ShapeDtypeStruct((B, 2 * D), jnp.float32),
    )(feat, start, end, W_ih, W_hh, bias)
